# baseline (speedup 1.0000x reference)
import functools

import jax
import jax.numpy as jnp
from jax import lax
from jax.experimental import pallas as pl
from jax.experimental.pallas import tpu as pltpu
from jax.experimental.pallas import tpu_sc as plsc

RANK = 32
NC, NS, L = 2, 16, 16
NW = NC * NS
IC = 128


def _sc_body(b_per_w, i_hbm, j_hbm, pT_hbm, qT_hbm, out_hbm,
             idx_i2, idx_j2, p_cols, q_cols, out_v, sem_p, sem_q):
    wid = lax.axis_index("s") * NC + lax.axis_index("c")
    base = wid * b_per_w
    nch = b_per_w // IC
    for c in range(nch):
        pltpu.sync_copy(i_hbm.at[pl.ds(base + c * IC, IC)], idx_i2.at[c])
        pltpu.sync_copy(j_hbm.at[pl.ds(base + c * IC, IC)], idx_j2.at[c])

    def fire_p(k, carry):
        for c in range(nch):
            pltpu.async_copy(pT_hbm.at[k].at[idx_i2.at[c]],
                             p_cols.at[k, pl.ds(c * IC, IC)], sem_p)
        return carry

    def fire_q(k, carry):
        for c in range(nch):
            pltpu.async_copy(qT_hbm.at[k].at[idx_j2.at[c]],
                             q_cols.at[k, pl.ds(c * IC, IC)], sem_q)
        return carry

    lax.fori_loop(0, RANK, fire_p, 0)
    lax.fori_loop(0, RANK + 1, fire_q, 0)

    def drain_p(t, carry):
        pltpu.make_async_copy(pT_hbm.at[0].at[idx_i2.at[0]],
                              p_cols.at[0, pl.ds(0, IC)], sem_p).wait()
        return carry

    def drain_q(t, carry):
        pltpu.make_async_copy(qT_hbm.at[0].at[idx_j2.at[0]],
                              q_cols.at[0, pl.ds(0, IC)], sem_q).wait()
        return carry

    lax.fori_loop(0, RANK * nch, drain_p, 0)
    lax.fori_loop(0, (RANK + 1) * nch, drain_q, 0)

    def group(g, carry):
        lanes = pl.ds(g * L, L)
        acc = q_cols[RANK, lanes]
        for k in range(RANK):
            acc = acc + p_cols[k, lanes] * q_cols[k, lanes]
        out_v[lanes] = 1.0 / (1.0 + jnp.exp(-acc))
        return carry

    lax.fori_loop(0, b_per_w // L, group, 0)
    pltpu.sync_copy(out_v, out_hbm.at[pl.ds(base, b_per_w)])


def kernel(i, j, p_weight, q_weight):
    b = i.shape[0]
    b_per_w = b // NW
    mesh = plsc.VectorSubcoreMesh(core_axis_name="c", subcore_axis_name="s")
    kfn = pl.kernel(
        functools.partial(_sc_body, b_per_w),
        out_type=jax.ShapeDtypeStruct((b,), jnp.float32),
        mesh=mesh,
        scratch_types=[
            pltpu.VMEM((b_per_w // IC, IC), jnp.int32),
            pltpu.VMEM((b_per_w // IC, IC), jnp.int32),
            pltpu.VMEM((RANK, b_per_w), jnp.float32),
            pltpu.VMEM((RANK + 1, b_per_w), jnp.float32),
            pltpu.VMEM((b_per_w,), jnp.float32),
            pltpu.SemaphoreType.DMA,
            pltpu.SemaphoreType.DMA,
        ],
        compiler_params=pltpu.CompilerParams(
            needs_layout_passes=False, use_tc_tiling_on_sc=False),
    )
    out = kfn(i.astype(jnp.int32), j.astype(jnp.int32),
              p_weight.T, q_weight.T)
    return out.reshape(-1, 1)


# speedup vs baseline: 8.3035x; 8.3035x over previous
import functools

import jax
import jax.numpy as jnp
from jax import lax
from jax.experimental import pallas as pl
from jax.experimental.pallas import tpu as pltpu
from jax.experimental.pallas import tpu_sc as plsc

RANK = 32
NC, NS, L = 2, 16, 16
NW = NC * NS
CHUNK = 256


def _sc_body(b_per_w, i_hbm, j_hbm, p_hbm, q_hbm, out_hbm,
             idx_i_v, idx_j_v, p_rows, q_rows, out_v, sem_p, sem_q):
    wid = lax.axis_index("s") * NC + lax.axis_index("c")
    base = wid * b_per_w
    pltpu.sync_copy(i_hbm.at[pl.ds(base, b_per_w)], idx_i_v)
    pltpu.sync_copy(j_hbm.at[pl.ds(base, b_per_w)], idx_j_v)

    for c in range(b_per_w // CHUNK):
        def fire(g, carry):
            vi = idx_i_v[pl.ds(c * CHUNK + g * L, L)]
            vj = idx_j_v[pl.ds(c * CHUNK + g * L, L)]
            for r in range(L):
                t = g * L + r
                pltpu.async_copy(p_hbm.at[vi[r]],
                                 p_rows.at[t, pl.ds(0, RANK)], sem_p)
                pltpu.async_copy(q_hbm.at[vj[r]],
                                 q_rows.at[t, pl.ds(0, RANK + 1)], sem_q)
            return carry

        lax.fori_loop(0, CHUNK // L, fire, 0)

        def drain(t, carry):
            pltpu.make_async_copy(p_hbm.at[0],
                                  p_rows.at[0, pl.ds(0, RANK)], sem_p).wait()
            pltpu.make_async_copy(q_hbm.at[0],
                                  q_rows.at[0, pl.ds(0, RANK + 1)], sem_q).wait()
            return carry

        lax.fori_loop(0, CHUNK, drain, 0)

        def group(g, carry):
            rows16 = g * L + lax.iota(jnp.int32, L)
            acc = plsc.load_gather(q_rows, [rows16, jnp.full((L,), RANK, jnp.int32)])
            for k in range(RANK):
                kk = jnp.full((L,), k, jnp.int32)
                pv = plsc.load_gather(p_rows, [rows16, kk])
                qv = plsc.load_gather(q_rows, [rows16, kk])
                acc = acc + pv * qv
            out_v[pl.ds(c * CHUNK + g * L, L)] = 1.0 / (1.0 + jnp.exp(-acc))
            return carry

        lax.fori_loop(0, CHUNK // L, group, 0)

    pltpu.sync_copy(out_v, out_hbm.at[pl.ds(base, b_per_w)])


def kernel(i, j, p_weight, q_weight):
    b = i.shape[0]
    b_per_w = b // NW
    mesh = plsc.VectorSubcoreMesh(core_axis_name="c", subcore_axis_name="s")
    kfn = pl.kernel(
        functools.partial(_sc_body, b_per_w),
        out_type=jax.ShapeDtypeStruct((b,), jnp.float32),
        mesh=mesh,
        scratch_types=[
            pltpu.VMEM((b_per_w,), jnp.int32),
            pltpu.VMEM((b_per_w,), jnp.int32),
            pltpu.VMEM((CHUNK, 128), jnp.float32),
            pltpu.VMEM((CHUNK, 128), jnp.float32),
            pltpu.VMEM((b_per_w,), jnp.float32),
            pltpu.SemaphoreType.DMA,
            pltpu.SemaphoreType.DMA,
        ],
        compiler_params=pltpu.CompilerParams(
            needs_layout_passes=False, use_tc_tiling_on_sc=True),
    )
    out = kfn(i.astype(jnp.int32), j.astype(jnp.int32), p_weight, q_weight)
    return out.reshape(-1, 1)
